# baseline (speedup 1.0000x reference)
import functools

import jax
import jax.numpy as jnp
from jax import lax
from jax.experimental import pallas as pl
from jax.experimental.pallas import tpu as pltpu


def _body(idx_ref, c_ref, d_ref, g_ref, br_ref, p_ref, *, bm, d, m):
    idx = idx_ref[0, :]
    onehot = (lax.broadcasted_iota(jnp.int32, (d, m), 0) == idx[None, :]).astype(
        jnp.float32
    )
    tc = lax.dot_general(
        c_ref[...], onehot, (((1,), (0,)), ((), ())),
        precision=lax.Precision.HIGHEST, preferred_element_type=jnp.float32,
    )
    td = lax.dot_general(
        d_ref[...], onehot, (((1,), (0,)), ((), ())),
        precision=lax.Precision.HIGHEST, preferred_element_type=jnp.float32,
    )
    lower = tc - td
    upper = tc + td
    max_lower = jnp.max(lower, axis=1, keepdims=True)
    mask = upper >= max_lower
    vol = 2.0 * td
    sel = jnp.where(mask, vol, 0.0)
    s = jnp.sum(sel, axis=1, keepdims=True)
    p = sel / s
    logits = jnp.where(mask, jnp.log(jnp.maximum(p, 1e-30)), -jnp.inf)
    z = logits + g_ref[...]
    res = jnp.argmax(z, axis=1)
    branch = lax.broadcasted_iota(jnp.int32, (bm, m), 1) == res[:, None]
    br_ref[...] = branch.astype(jnp.uint8)
    p_ref[...] = jnp.where(branch, p, 0.0)


@functools.partial(jax.jit, static_argnames=("interpret",))
def kernel(c, delta, arg_idx, interpret=False):
    b, d = c.shape
    m = arg_idx.shape[0]
    bm = 512
    g = jax.random.gumbel(jax.random.key(42), (b, m), jnp.float32)
    idx2d = arg_idx.astype(jnp.int32).reshape(1, m)
    grid = (b // bm,)
    br_u8, p_out = pl.pallas_call(
        functools.partial(_body, bm=bm, d=d, m=m),
        grid=grid,
        in_specs=[
            pl.BlockSpec((1, m), lambda i: (0, 0)),
            pl.BlockSpec((bm, d), lambda i: (i, 0)),
            pl.BlockSpec((bm, d), lambda i: (i, 0)),
            pl.BlockSpec((bm, m), lambda i: (i, 0)),
        ],
        out_specs=[
            pl.BlockSpec((bm, m), lambda i: (i, 0)),
            pl.BlockSpec((bm, m), lambda i: (i, 0)),
        ],
        out_shape=[
            jax.ShapeDtypeStruct((b, m), jnp.uint8),
            jax.ShapeDtypeStruct((b, m), jnp.float32),
        ],
        interpret=interpret,
    )(idx2d, c, delta, g)
    return br_u8.astype(jnp.bool_), p_out
